# baseline (speedup 1.0000x reference)
import functools

import jax
import jax.numpy as jnp
from jax import lax
from jax.experimental import pallas as pl
from jax.experimental.pallas import tpu as pltpu
from jax.experimental.pallas import tpu_sc as plsc

_CHUNK = 128
_LANES = 16


@functools.partial(jax.jit, static_argnames=("d",))
def _gather_rows(table, dim_vec, idx, d):
    info = plsc.get_sparse_core_info()
    nw = info.num_cores * info.num_subcores
    b = idx.shape[0]
    chunks_per_w = b // (nw * _CHUNK)
    b_per_w = chunks_per_w * _CHUNK

    mesh = plsc.VectorSubcoreMesh(core_axis_name="c", subcore_axis_name="s")

    @functools.partial(
        pl.kernel,
        mesh=mesh,
        out_type=jax.ShapeDtypeStruct((b, d), jnp.float32),
        scratch_types=[
            pltpu.VMEM((_LANES,), jnp.int32),
            pltpu.VMEM((b_per_w,), jnp.int32),
            pltpu.VMEM((b_per_w, d), jnp.float32),
            pltpu.SemaphoreType.DMA((chunks_per_w,)),
            pltpu.SemaphoreType.DMA,
        ],
    )
    def k(table_hbm, dim_hbm, idx_hbm, out_hbm, dim_v, idx_v, rows_v, gsem,
          wsem):
        wid = lax.axis_index("s") * info.num_cores + lax.axis_index("c")
        base = wid * b_per_w
        pltpu.sync_copy(dim_hbm, dim_v)
        pltpu.sync_copy(idx_hbm.at[pl.ds(base, b_per_w)], idx_v)
        dv = dim_v[...]
        for j in range(chunks_per_w):
            for t in range(_CHUNK // _LANES):
                s = j * _CHUNK + t * _LANES
                idx_v[pl.ds(s, _LANES)] = idx_v[pl.ds(s, _LANES)] + dv
            pltpu.async_copy(
                table_hbm.at[idx_v.at[pl.ds(j * _CHUNK, _CHUNK)]],
                rows_v.at[pl.ds(j * _CHUNK, _CHUNK)], gsem.at[j])
        for j in range(chunks_per_w):
            pltpu.make_async_copy(
                table_hbm.at[idx_v.at[pl.ds(j * _CHUNK, _CHUNK)]],
                rows_v.at[pl.ds(j * _CHUNK, _CHUNK)], gsem.at[j]).wait()
            pltpu.async_copy(rows_v.at[pl.ds(j * _CHUNK, _CHUNK)],
                             out_hbm.at[pl.ds(base + j * _CHUNK, _CHUNK)],
                             wsem)
        pltpu.make_async_copy(rows_v, out_hbm.at[pl.ds(base, b_per_w)],
                              wsem).wait()

    return k(table, dim_vec, idx)


def kernel(input, dim, index):
    d = input.shape[1]
    idx = index.astype(jnp.int32)
    dim_vec = jnp.full((_LANES,), dim, dtype=jnp.int32)
    return _gather_rows(input, dim_vec, idx, d=d)


# speedup vs baseline: 1.0296x; 1.0296x over previous
import functools

import jax
import jax.numpy as jnp
from jax import lax
from jax.experimental import pallas as pl
from jax.experimental.pallas import tpu as pltpu
from jax.experimental.pallas import tpu_sc as plsc

_CHUNK = 128
_LANES = 16


@functools.partial(jax.jit, static_argnames=("d",))
def _gather_rows(table, dim_vec, idx, d):
    info = plsc.get_sparse_core_info()
    nw = info.num_cores * info.num_subcores
    b = idx.shape[0]
    chunks_per_w = b // (nw * _CHUNK)
    b_per_w = chunks_per_w * _CHUNK

    mesh = plsc.VectorSubcoreMesh(core_axis_name="c", subcore_axis_name="s")

    @functools.partial(
        pl.kernel,
        mesh=mesh,
        out_type=jax.ShapeDtypeStruct((b, d), jnp.float32),
        scratch_types=[
            pltpu.VMEM((_LANES,), jnp.int32),
            pltpu.VMEM((b_per_w,), jnp.int32),
            pltpu.VMEM((b_per_w, d), jnp.float32),
            pltpu.SemaphoreType.DMA,
            pltpu.SemaphoreType.DMA,
        ],
    )
    def k(table_hbm, dim_hbm, idx_hbm, out_hbm, dim_v, idx_v, rows_v, dsem,
          gsem):
        wid = lax.axis_index("s") * info.num_cores + lax.axis_index("c")
        base = wid * b_per_w
        pltpu.async_copy(dim_hbm, dim_v, dsem)
        pltpu.sync_copy(idx_hbm.at[pl.ds(base, b_per_w)], idx_v)
        pltpu.make_async_copy(dim_hbm, dim_v, dsem).wait()
        dv = dim_v[...]
        for t in range(b_per_w // _LANES):
            s = t * _LANES
            idx_v[pl.ds(s, _LANES)] = idx_v[pl.ds(s, _LANES)] + dv
        for j in range(chunks_per_w):
            pltpu.async_copy(
                table_hbm.at[idx_v.at[pl.ds(j * _CHUNK, _CHUNK)]],
                rows_v.at[pl.ds(j * _CHUNK, _CHUNK)], gsem)
        for j in range(chunks_per_w):
            pltpu.make_async_copy(
                table_hbm.at[idx_v.at[pl.ds(j * _CHUNK, _CHUNK)]],
                rows_v.at[pl.ds(j * _CHUNK, _CHUNK)], gsem).wait()
        pltpu.sync_copy(rows_v, out_hbm.at[pl.ds(base, b_per_w)])

    return k(table, dim_vec, idx)


def kernel(input, dim, index):
    d = input.shape[1]
    idx = index.astype(jnp.int32)
    dim_vec = jnp.full((_LANES,), dim, dtype=jnp.int32)
    return _gather_rows(input, dim_vec, idx, d=d)


# speedup vs baseline: 1.0607x; 1.0301x over previous
import functools

import jax
import jax.numpy as jnp
from jax import lax
from jax.experimental import pallas as pl
from jax.experimental.pallas import tpu as pltpu
from jax.experimental.pallas import tpu_sc as plsc

_CHUNK = 128


@functools.partial(jax.jit, static_argnames=("d",))
def _gather_rows(table, idx2d, d):
    info = plsc.get_sparse_core_info()
    nw = info.num_cores * info.num_subcores
    b = idx2d.shape[0] * idx2d.shape[1]
    chunks_per_w = b // (nw * _CHUNK)
    b_per_w = chunks_per_w * _CHUNK

    mesh = plsc.VectorSubcoreMesh(core_axis_name="c", subcore_axis_name="s")

    @functools.partial(
        pl.kernel,
        mesh=mesh,
        out_type=jax.ShapeDtypeStruct((b, d), jnp.float32),
        scratch_types=[
            pltpu.VMEM((chunks_per_w, _CHUNK), jnp.int32),
            pltpu.VMEM((b_per_w, d), jnp.float32),
            pltpu.SemaphoreType.DMA,
        ],
    )
    def k(table_hbm, idx_hbm, out_hbm, idx_v, rows_v, sem):
        wid = lax.axis_index("s") * info.num_cores + lax.axis_index("c")
        pltpu.sync_copy(idx_hbm.at[pl.ds(wid * chunks_per_w, chunks_per_w)],
                        idx_v)
        for j in range(chunks_per_w):
            pltpu.async_copy(table_hbm.at[idx_v.at[j]],
                             rows_v.at[pl.ds(j * _CHUNK, _CHUNK)], sem)
        for j in range(chunks_per_w):
            pltpu.make_async_copy(table_hbm.at[idx_v.at[j]],
                                  rows_v.at[pl.ds(j * _CHUNK, _CHUNK)],
                                  sem).wait()
        pltpu.sync_copy(rows_v, out_hbm.at[pl.ds(wid * b_per_w, b_per_w)])

    return k(table, idx2d)


def kernel(input, dim, index):
    b = index.shape[0]
    d = input.shape[1]
    idx = (index + dim).astype(jnp.int32).reshape(b // _CHUNK, _CHUNK)
    return _gather_rows(input, idx, d=d)
